# baseline (speedup 1.0000x reference)
import functools
import math

import jax
import jax.numpy as jnp
from jax import lax
from jax.experimental import pallas as pl
from jax.experimental.pallas import tpu as pltpu
from jax.experimental.pallas import tpu_sc as plsc

VOCAB = 1000000
D_MODEL = 64
SCALE = math.sqrt(D_MODEL)

B, S = 4096, 200
N = B * S
NC, NS, L = 2, 16, 16
NW = NC * NS
N_PER_W = N // NW
C = 512
G = N_PER_W // C

_mesh = plsc.VectorSubcoreMesh(core_axis_name="c", subcore_axis_name="s")


@functools.partial(
    pl.kernel,
    mesh=_mesh,
    compiler_params=pltpu.CompilerParams(use_tc_tiling_on_sc=False),
    out_type=jax.ShapeDtypeStruct((N, D_MODEL), jnp.float32),
    scratch_types=[
        pltpu.VMEM((C,), jnp.int32),
        pltpu.VMEM((C, D_MODEL), jnp.float32),
        pltpu.SemaphoreType.DMA,
    ],
)
def _emb_lookup(idx_hbm, table_hbm, out_hbm, idx_v, rows_v, sem):
    wid = lax.axis_index("s") * NC + lax.axis_index("c")
    base = wid * N_PER_W

    def chunk_body(g, carry):
        off = base + g * C
        pltpu.sync_copy(idx_hbm.at[pl.ds(off, C)], idx_v)
        pltpu.async_copy(table_hbm.at[idx_v], rows_v, sem).wait()

        def row_body(r, c2):
            for j in range(D_MODEL // L):
                sl = pl.ds(j * L, L)
                rows_v[r, sl] = rows_v[r, sl] * SCALE
            return c2

        lax.fori_loop(0, C, row_body, 0, unroll=4)
        pltpu.sync_copy(rows_v, out_hbm.at[pl.ds(off, C)])
        return carry

    lax.fori_loop(0, G, chunk_body, 0)


def kernel(x, table):
    out = _emb_lookup(x.reshape(N), table)
    return out.reshape(B, S, D_MODEL)


# speedup vs baseline: 1.0923x; 1.0923x over previous
import functools
import math

import jax
import jax.numpy as jnp
from jax import lax
from jax.experimental import pallas as pl
from jax.experimental.pallas import tpu as pltpu
from jax.experimental.pallas import tpu_sc as plsc

VOCAB = 1000000
D_MODEL = 64
SCALE = math.sqrt(D_MODEL)

B, S = 4096, 200
N = B * S
NC, NS, L = 2, 16, 16
NW = NC * NS
N_PER_W = N // NW
C = 800
G = N_PER_W // C
H = G // 2

_mesh = plsc.VectorSubcoreMesh(core_axis_name="c", subcore_axis_name="s")


@functools.partial(
    pl.kernel,
    mesh=_mesh,
    compiler_params=pltpu.CompilerParams(use_tc_tiling_on_sc=False),
    out_type=jax.ShapeDtypeStruct((N, D_MODEL), jnp.float32),
    scratch_types=[
        pltpu.VMEM((N_PER_W,), jnp.int32),
        pltpu.VMEM((C, D_MODEL), jnp.float32),
        pltpu.VMEM((C, D_MODEL), jnp.float32),
        pltpu.SemaphoreType.DMA,
        pltpu.SemaphoreType.DMA,
        pltpu.SemaphoreType.DMA,
        pltpu.SemaphoreType.DMA,
    ],
)
def _emb_lookup(idx_hbm, table_hbm, out_hbm, idx_v, rows0, rows1,
                gsem0, gsem1, osem0, osem1):
    wid = lax.axis_index("s") * NC + lax.axis_index("c")
    base = wid * N_PER_W

    pltpu.sync_copy(idx_hbm.at[pl.ds(base, N_PER_W)], idx_v)

    rows = (rows0, rows1)
    gsem = (gsem0, gsem1)
    osem = (osem0, osem1)

    def gather(g, b):
        pltpu.async_copy(table_hbm.at[idx_v.at[pl.ds(g * C, C)]], rows[b],
                         gsem[b])

    def gather_wait(b):
        pltpu.make_async_copy(table_hbm.at[idx_v.at[pl.ds(0, C)]], rows[b],
                              gsem[b]).wait()

    def scatter(g, b):
        pltpu.async_copy(rows[b], out_hbm.at[pl.ds(base + g * C, C)], osem[b])

    def scatter_wait(b):
        pltpu.make_async_copy(rows[b], out_hbm.at[pl.ds(0, C)], osem[b]).wait()

    def scale(b):
        buf = rows[b]

        def row_body(r, carry):
            for j in range(D_MODEL // L):
                sl = pl.ds(j * L, L)
                buf[r, sl] = buf[r, sl] * SCALE
            return carry

        lax.fori_loop(0, C, row_body, 0, unroll=8)

    gather(0, 0)

    def step(h, carry):
        gather_wait(0)

        @pl.when(h > 0)
        def _():
            scatter_wait(1)

        gather(2 * h + 1, 1)
        scale(0)
        scatter(2 * h, 0)

        gather_wait(1)
        scatter_wait(0)

        @pl.when(h < H - 1)
        def _():
            gather(2 * h + 2, 0)

        scale(1)
        scatter(2 * h + 1, 1)
        return carry

    lax.fori_loop(0, H, step, 0)
    scatter_wait(1)


def kernel(x, table):
    out = _emb_lookup(x.reshape(N), table)
    return out.reshape(B, S, D_MODEL)
